# Initial kernel scaffold; baseline (speedup 1.0000x reference)
import jax
import jax.numpy as jnp
from jax.experimental import pallas as pl


def kernel(x, edge_index, Wd1, bd1, Wd2, bd2, Wd3, bd3, Wg1, bg1, Wg2, bg2, Wg3, bg3):
    raise NotImplementedError("write your pallas kernel here")


# speedup vs baseline: 12.5273x; 12.5273x over previous
import functools

import jax
import jax.numpy as jnp
import numpy as np
from jax import lax
from jax.experimental import pallas as pl
from jax.experimental.pallas import tpu as pltpu
from jax.experimental.pallas import tpu_sc as plsc

N_NODES = 10000
TIMESTEPS = 50
HIDDEN = 64
HORIZON = 5

NC = 2
NS = 16
NW = NC * NS
K = 128

N_PAD = 10240
RPT = N_PAD // NS

N_SUB = 6
H_STEP = 1.25 / N_SUB
SQH = float(np.sqrt(H_STEP))
N_STEPS = (HORIZON - 1) * N_SUB


def _make_agg(F, n_chunks):
  mesh = plsc.VectorSubcoreMesh(core_axis_name="c", subcore_axis_name="s")

  params = None if F == 128 else pltpu.CompilerParams(use_tc_tiling_on_sc=False)

  @functools.partial(
      pl.kernel,
      out_type=jax.ShapeDtypeStruct((NC, N_PAD, F), jnp.float32),
      mesh=mesh,
      compiler_params=params,
      scratch_types=[
          pltpu.VMEM((K,), jnp.int32),
          pltpu.VMEM((K,), jnp.int32),
          pltpu.VMEM((K, F), jnp.float32),
          pltpu.VMEM_SHARED((N_PAD, F), jnp.float32),
          pltpu.SemaphoreType.DMA,
      ],
  )
  def agg(v_hbm, srcs_hbm, dsts_hbm, zeros_hbm, out_hbm,
          src_v, dst_v, rows_v, accum_sh, sem):
    c = lax.axis_index("c")
    sid = lax.axis_index("s")
    wid = sid * NC + c
    row0 = sid * RPT

    pltpu.sync_copy(zeros_hbm, accum_sh.at[pl.ds(row0, RPT)])
    plsc.subcore_barrier()

    @pl.loop(0, n_chunks)
    def _(j):
      pltpu.sync_copy(srcs_hbm.at[wid, j], src_v)
      pltpu.sync_copy(dsts_hbm.at[wid, j], dst_v)
      pltpu.async_copy(v_hbm.at[src_v], rows_v, sem).wait()
      pltpu.sync_copy(rows_v, accum_sh.at[dst_v], add=True)

    plsc.subcore_barrier()
    pltpu.sync_copy(accum_sh.at[pl.ds(row0, RPT)],
                    out_hbm.at[c, pl.ds(row0, RPT)])

  return agg


def _tc_prep_body(degp_ref, x0_ref, dis_ref, v0_ref):
  deg = degp_ref[0, :, 0:1] + degp_ref[1, :, 0:1]
  dis = jnp.where(deg > 0, 1.0 / jnp.sqrt(deg), 0.0)
  dis_ref[...] = dis
  v0_ref[:, 0:1] = dis * x0_ref[...]
  v0_ref[:, 1:] = jnp.zeros_like(v0_ref[:, 1:])


def _tc_a_body(p_ref, dis_ref, wd1_ref, bd1_ref, wg1_ref, bg1_ref,
               wd2_ref, wg2_ref, out_ref):
  dis = dis_ref[...]
  a = (p_ref[0, :, 0:1] + p_ref[1, :, 0:1]) * dis
  h1d = jnp.tanh(a * wd1_ref[...] + bd1_ref[...])
  h1g = jnp.tanh(a * wg1_ref[...] + bg1_ref[...])
  ud = jnp.dot(h1d, wd2_ref[...], preferred_element_type=jnp.float32)
  ug = jnp.dot(h1g, wg2_ref[...], preferred_element_type=jnp.float32)
  out_ref[:, :HIDDEN] = dis * ud
  out_ref[:, HIDDEN:] = dis * ug


def _tc_b_body(p_ref, dis_ref, bd2_ref, bg2_ref, wd3_ref, wg3_ref, out_ref):
  dis = dis_ref[...]
  agg = (p_ref[0] + p_ref[1]) * dis
  h2d = jnp.tanh(agg[:, :HIDDEN] + bd2_ref[...])
  h2g = jnp.tanh(agg[:, HIDDEN:] + bg2_ref[...])
  zd = jnp.dot(h2d, wd3_ref[...], preferred_element_type=jnp.float32)
  zg = jnp.dot(h2g, wg3_ref[...], preferred_element_type=jnp.float32)
  out_ref[:, 0:1] = dis * zd
  out_ref[:, 1:2] = dis * zg
  out_ref[:, 2:] = jnp.zeros_like(out_ref[:, 2:])


def _tc_c_body(p_ref, s_ref, dis_ref, dw_ref, bd3_ref, bg3_ref,
               snew_ref, v0_ref):
  dis = dis_ref[...]
  drift = (p_ref[0, :, 0:1] + p_ref[1, :, 0:1]) * dis + bd3_ref[...]
  diffv = jnp.abs((p_ref[0, :, 1:2] + p_ref[1, :, 1:2]) * dis + bg3_ref[...])
  sn = s_ref[...] + drift * H_STEP + diffv * SQH * dw_ref[...]
  snew_ref[...] = sn
  v0_ref[:, 0:1] = dis * sn
  v0_ref[:, 1:] = jnp.zeros_like(v0_ref[:, 1:])


def _tc_call(body, out_shapes, *args, interpret=False):
  return pl.pallas_call(
      body,
      out_shape=out_shapes,
      interpret=interpret,
  )(*args)


def kernel(x, edge_index, Wd1, bd1, Wd2, bd2, Wd3, bd3,
           Wg1, bg1, Wg2, bg2, Wg3, bg3):
  n = x.shape[1]
  x0 = x[0, :, -1:]

  e = edge_index.shape[1]
  loops = jnp.arange(n, dtype=jnp.int32)
  src = jnp.concatenate([edge_index[0].astype(jnp.int32), loops])
  dst = jnp.concatenate([edge_index[1].astype(jnp.int32), loops])
  e_tot = e + n
  n_chunks = -(-e_tot // (NW * K))
  e_pad = NW * K * n_chunks
  src = jnp.concatenate([src, jnp.full((e_pad - e_tot,), n, jnp.int32)])
  dst = jnp.concatenate([dst, jnp.full((e_pad - e_tot,), n, jnp.int32)])
  srcs = src.reshape(NW, n_chunks, K)
  dsts = dst.reshape(NW, n_chunks, K)

  zeros16 = jnp.zeros((RPT, 16), jnp.float32)
  zeros128 = jnp.zeros((RPT, 128), jnp.float32)
  ones_tbl = jnp.zeros((N_PAD, 16), jnp.float32).at[:n, 0].set(1.0)
  x0p = jnp.zeros((N_PAD, 1), jnp.float32).at[:n].set(x0)

  nkey = jax.random.key(1234)
  dws = jnp.stack([
      jax.random.normal(jax.random.fold_in(nkey, s), (n, 1), jnp.float32)
      for s in range(N_STEPS)
  ])
  dws = jnp.concatenate(
      [dws, jnp.zeros((N_STEPS, N_PAD - n, 1), jnp.float32)], axis=1)

  agg16 = _make_agg(16, n_chunks)
  agg128 = _make_agg(128, n_chunks)

  b1d = bd1.reshape(1, HIDDEN)
  b1g = bg1.reshape(1, HIDDEN)
  b2d = bd2.reshape(1, HIDDEN)
  b2g = bg2.reshape(1, HIDDEN)
  b3d = bd3.reshape(1, 1)
  b3g = bg3.reshape(1, 1)

  degp = agg16(ones_tbl, srcs, dsts, zeros16)
  dis, v0 = _tc_call(
      _tc_prep_body,
      (jax.ShapeDtypeStruct((N_PAD, 1), jnp.float32),
       jax.ShapeDtypeStruct((N_PAD, 16), jnp.float32)),
      degp, x0p)

  def step(carry, dw):
    s, v0 = carry
    p0 = agg16(v0, srcs, dsts, zeros16)
    v1 = _tc_call(
        _tc_a_body,
        jax.ShapeDtypeStruct((N_PAD, 2 * HIDDEN), jnp.float32),
        p0, dis, Wd1, b1d, Wg1, b1g, Wd2, Wg2)
    p1 = agg128(v1, srcs, dsts, zeros128)
    v2 = _tc_call(
        _tc_b_body,
        jax.ShapeDtypeStruct((N_PAD, 16), jnp.float32),
        p1, dis, b2d, b2g, Wd3, Wg3)
    p2 = agg16(v2, srcs, dsts, zeros16)
    sn, v0n = _tc_call(
        _tc_c_body,
        (jax.ShapeDtypeStruct((N_PAD, 1), jnp.float32),
         jax.ShapeDtypeStruct((N_PAD, 16), jnp.float32)),
        p2, s, dis, dw, b3d, b3g)
    return (sn, v0n), sn

  (_, _), ys = lax.scan(step, (x0p, v0), dws)

  keep = ys[jnp.array([N_SUB - 1, 2 * N_SUB - 1, 3 * N_SUB - 1,
                       4 * N_SUB - 1])]
  preds = jnp.concatenate([x0p[None], keep], axis=0)[:, :n, 0]
  return preds.T[None]


# speedup vs baseline: 20.4325x; 1.6310x over previous
import functools

import jax
import jax.numpy as jnp
import numpy as np
from jax import lax
from jax.experimental import pallas as pl
from jax.experimental.pallas import tpu as pltpu
from jax.experimental.pallas import tpu_sc as plsc

N_NODES = 10000
TIMESTEPS = 50
HIDDEN = 64
HORIZON = 5

NC = 2
NS = 16
NW = NC * NS

N_PAD = 10240
RPT = N_PAD // NS

N_SUB = 6
H_STEP = 1.25 / N_SUB
SQH = float(np.sqrt(H_STEP))
N_STEPS = (HORIZON - 1) * N_SUB


NBUF = 3


def _make_agg(F, K, n_chunks):
  assert n_chunks % NBUF == 0
  mesh = plsc.VectorSubcoreMesh(core_axis_name="c", subcore_axis_name="s")

  params = None if F == 128 else pltpu.CompilerParams(use_tc_tiling_on_sc=False)

  @functools.partial(
      pl.kernel,
      out_type=jax.ShapeDtypeStruct((NC, N_PAD, F), jnp.float32),
      mesh=mesh,
      compiler_params=params,
      scratch_types=[
          pltpu.VMEM((n_chunks, 2, K), jnp.int32),
          [pltpu.VMEM((K, F), jnp.float32)] * NBUF,
          pltpu.VMEM_SHARED((N_PAD, F), jnp.float32),
          [pltpu.SemaphoreType.DMA] * NBUF,
          [pltpu.SemaphoreType.DMA] * NBUF,
      ],
  )
  def agg(v_hbm, sd_hbm, zeros_hbm, out_hbm,
          slab_v, bufs, accum_sh, gsems, ssems):
    c = lax.axis_index("c")
    sid = lax.axis_index("s")
    wid = sid * NC + c
    row0 = sid * RPT

    pltpu.sync_copy(sd_hbm.at[wid], slab_v)
    for b in range(NBUF):
      pltpu.async_copy(v_hbm.at[slab_v.at[b, 0]], bufs[b], gsems[b])
    pltpu.sync_copy(zeros_hbm, accum_sh.at[pl.ds(row0, RPT)])
    plsc.subcore_barrier()

    @pl.loop(0, n_chunks, step=NBUF)
    def _(j0):
      for b in range(NBUF):
        j = j0 + b
        pltpu.make_async_copy(
            v_hbm.at[slab_v.at[j, 0]], bufs[b], gsems[b]).wait()
        sdesc = pltpu.make_async_copy(
            bufs[b], accum_sh.at[slab_v.at[j, 1]], ssems[b])
        sdesc.start(add=True)
        sdesc.wait()

        @pl.when(j + NBUF < n_chunks)
        def _():
          pltpu.async_copy(
              v_hbm.at[slab_v.at[j + NBUF, 0]], bufs[b], gsems[b])

    plsc.subcore_barrier()
    pltpu.sync_copy(accum_sh.at[pl.ds(row0, RPT)],
                    out_hbm.at[c, pl.ds(row0, RPT)])

  return agg


def _tc_prep_body(degp_ref, x0_ref, dis_ref, v0_ref):
  deg = degp_ref[0, :, 0:1] + degp_ref[1, :, 0:1]
  dis = jnp.where(deg > 0, 1.0 / jnp.sqrt(deg), 0.0)
  dis_ref[...] = dis
  v0_ref[:, 0:1] = dis * x0_ref[...]
  v0_ref[:, 1:] = jnp.zeros_like(v0_ref[:, 1:])


def _tc_a_body(p_ref, dis_ref, wd1_ref, bd1_ref, wg1_ref, bg1_ref,
               wd2_ref, wg2_ref, out_ref):
  dis = dis_ref[...]
  a = (p_ref[0, :, 0:1] + p_ref[1, :, 0:1]) * dis
  h1d = jnp.tanh(a * wd1_ref[...] + bd1_ref[...])
  h1g = jnp.tanh(a * wg1_ref[...] + bg1_ref[...])
  ud = jnp.dot(h1d, wd2_ref[...], preferred_element_type=jnp.float32)
  ug = jnp.dot(h1g, wg2_ref[...], preferred_element_type=jnp.float32)
  out_ref[:, :HIDDEN] = dis * ud
  out_ref[:, HIDDEN:] = dis * ug


def _tc_b_body(p_ref, dis_ref, bd2_ref, bg2_ref, wd3_ref, wg3_ref, out_ref):
  dis = dis_ref[...]
  agg = (p_ref[0] + p_ref[1]) * dis
  h2d = jnp.tanh(agg[:, :HIDDEN] + bd2_ref[...])
  h2g = jnp.tanh(agg[:, HIDDEN:] + bg2_ref[...])
  zd = jnp.dot(h2d, wd3_ref[...], preferred_element_type=jnp.float32)
  zg = jnp.dot(h2g, wg3_ref[...], preferred_element_type=jnp.float32)
  out_ref[:, 0:1] = dis * zd
  out_ref[:, 1:2] = dis * zg
  out_ref[:, 2:] = jnp.zeros_like(out_ref[:, 2:])


def _tc_c_body(p_ref, s_ref, dis_ref, dw_ref, bd3_ref, bg3_ref,
               snew_ref, v0_ref):
  dis = dis_ref[...]
  drift = (p_ref[0, :, 0:1] + p_ref[1, :, 0:1]) * dis + bd3_ref[...]
  diffv = jnp.abs((p_ref[0, :, 1:2] + p_ref[1, :, 1:2]) * dis + bg3_ref[...])
  sn = s_ref[...] + drift * H_STEP + diffv * SQH * dw_ref[...]
  snew_ref[...] = sn
  v0_ref[:, 0:1] = dis * sn
  v0_ref[:, 1:] = jnp.zeros_like(v0_ref[:, 1:])


def _tc_call(body, out_shapes, *args, interpret=False):
  return pl.pallas_call(
      body,
      out_shape=out_shapes,
      interpret=interpret,
  )(*args)


def kernel(x, edge_index, Wd1, bd1, Wd2, bd2, Wd3, bd3,
           Wg1, bg1, Wg2, bg2, Wg3, bg3):
  n = x.shape[1]
  x0 = x[0, :, -1:]

  e = edge_index.shape[1]
  loops = jnp.arange(n, dtype=jnp.int32)
  src = jnp.concatenate([edge_index[0].astype(jnp.int32), loops])
  dst = jnp.concatenate([edge_index[1].astype(jnp.int32), loops])
  e_tot = e + n
  k16, k128 = 128, 64
  per_w = NW * k16 * NBUF
  n_chunks16 = -(-e_tot // (NW * k16))
  n_chunks16 = -(-n_chunks16 // NBUF) * NBUF
  e_pad = NW * k16 * n_chunks16
  n_chunks128 = e_pad // (NW * k128)
  src = jnp.concatenate([src, jnp.full((e_pad - e_tot,), n, jnp.int32)])
  dst = jnp.concatenate([dst, jnp.full((e_pad - e_tot,), n, jnp.int32)])
  sd16 = jnp.stack([src.reshape(NW, n_chunks16, k16),
                    dst.reshape(NW, n_chunks16, k16)], axis=2)
  sd128 = jnp.stack([src.reshape(NW, n_chunks128, k128),
                     dst.reshape(NW, n_chunks128, k128)], axis=2)

  zeros16 = jnp.zeros((RPT, 16), jnp.float32)
  zeros128 = jnp.zeros((RPT, 128), jnp.float32)
  ones_tbl = jnp.zeros((N_PAD, 16), jnp.float32).at[:n, 0].set(1.0)
  x0p = jnp.zeros((N_PAD, 1), jnp.float32).at[:n].set(x0)

  nkey = jax.random.key(1234)
  dws = jnp.stack([
      jax.random.normal(jax.random.fold_in(nkey, s), (n, 1), jnp.float32)
      for s in range(N_STEPS)
  ])
  dws = jnp.concatenate(
      [dws, jnp.zeros((N_STEPS, N_PAD - n, 1), jnp.float32)], axis=1)

  agg16 = _make_agg(16, k16, n_chunks16)
  agg128 = _make_agg(128, k128, n_chunks128)

  b1d = bd1.reshape(1, HIDDEN)
  b1g = bg1.reshape(1, HIDDEN)
  b2d = bd2.reshape(1, HIDDEN)
  b2g = bg2.reshape(1, HIDDEN)
  b3d = bd3.reshape(1, 1)
  b3g = bg3.reshape(1, 1)

  degp = agg16(ones_tbl, sd16, zeros16)
  dis, v0 = _tc_call(
      _tc_prep_body,
      (jax.ShapeDtypeStruct((N_PAD, 1), jnp.float32),
       jax.ShapeDtypeStruct((N_PAD, 16), jnp.float32)),
      degp, x0p)

  def step(carry, dw):
    s, v0 = carry
    p0 = agg16(v0, sd16, zeros16)
    v1 = _tc_call(
        _tc_a_body,
        jax.ShapeDtypeStruct((N_PAD, 2 * HIDDEN), jnp.float32),
        p0, dis, Wd1, b1d, Wg1, b1g, Wd2, Wg2)
    p1 = agg128(v1, sd128, zeros128)
    v2 = _tc_call(
        _tc_b_body,
        jax.ShapeDtypeStruct((N_PAD, 16), jnp.float32),
        p1, dis, b2d, b2g, Wd3, Wg3)
    p2 = agg16(v2, sd16, zeros16)
    sn, v0n = _tc_call(
        _tc_c_body,
        (jax.ShapeDtypeStruct((N_PAD, 1), jnp.float32),
         jax.ShapeDtypeStruct((N_PAD, 16), jnp.float32)),
        p2, s, dis, dw, b3d, b3g)
    return (sn, v0n), sn

  (_, _), ys = lax.scan(step, (x0p, v0), dws)

  keep = ys[jnp.array([N_SUB - 1, 2 * N_SUB - 1, 3 * N_SUB - 1,
                       4 * N_SUB - 1])]
  preds = jnp.concatenate([x0p[None], keep], axis=0)[:, :n, 0]
  return preds.T[None]
